# baseline (speedup 1.0000x reference)
import functools

import jax
import jax.numpy as jnp
from jax import lax
from jax.experimental import pallas as pl
from jax.experimental.pallas import tpu as pltpu
from jax.experimental.pallas import tpu_sc as plsc

N = 10000
H = 128
NUM_WORKERS = 32
CHUNK = 128
ROW_BLK = 1000
AGG_ROWS = 10240
ZERO_BLK = 128


def _gelu(v):
    return 0.5 * v * (1.0 + lax.erf(v * 0.7071067811865475))


def _mm(a, b_t):
    return lax.dot_general(a, b_t, (((1,), (1,)), ((), ())),
                           preferred_element_type=jnp.float32)


def _emb1_body(x_ref, w_ref, b_ref, o_ref):
    o_ref[...] = _gelu(_mm(x_ref[...], w_ref[...]) + b_ref[...])


def _emb1(x, w, b):
    n = x.shape[0]
    grid = (n // ROW_BLK,)
    return pl.pallas_call(
        _emb1_body,
        grid=grid,
        in_specs=[
            pl.BlockSpec((ROW_BLK, H), lambda i: (i, 0)),
            pl.BlockSpec((H, H), lambda i: (0, 0)),
            pl.BlockSpec((1, H), lambda i: (0, 0)),
        ],
        out_specs=pl.BlockSpec((ROW_BLK, H), lambda i: (i, 0)),
        out_shape=jax.ShapeDtypeStruct((n, H), jnp.float32),
    )(x, w, b.reshape(1, H))


def _layer_body(h_ref, p0_ref, p1_ref, ws_ref, wn_ref, o_ref):
    h = h_ref[...]
    agg = p0_ref[...] + p1_ref[...]
    o_ref[...] = _gelu(_mm(h, ws_ref[...]) + _mm(agg, wn_ref[...])) + h


def _layer(h, p0, p1, ws, wn):
    n = h.shape[0]
    grid = (n // ROW_BLK,)
    blk = pl.BlockSpec((ROW_BLK, H), lambda i: (i, 0))
    wblk = pl.BlockSpec((H, H), lambda i: (0, 0))
    return pl.pallas_call(
        _layer_body,
        grid=grid,
        in_specs=[blk, blk, blk, wblk, wblk],
        out_specs=blk,
        out_shape=jax.ShapeDtypeStruct((n, H), jnp.float32),
    )(h, p0, p1, ws, wn)


def _final_body(h_ref, p0_ref, p1_ref, ws_ref, wn_ref, w2_ref, b2_ref, o_ref):
    h = h_ref[...]
    agg = p0_ref[...] + p1_ref[...]
    h2 = _gelu(_mm(h, ws_ref[...]) + _mm(agg, wn_ref[...])) + h
    proj = jnp.sum(h2 * w2_ref[...], axis=1, keepdims=True) + b2_ref[0, 0]
    o_ref[...] = jnp.tanh(proj)


def _final(h, p0, p1, ws, wn, w2, b2):
    n = h.shape[0]
    grid = (n // ROW_BLK,)
    blk = pl.BlockSpec((ROW_BLK, H), lambda i: (i, 0))
    wblk = pl.BlockSpec((H, H), lambda i: (0, 0))
    return pl.pallas_call(
        _final_body,
        grid=grid,
        in_specs=[blk, blk, blk, wblk, wblk,
                  pl.BlockSpec((1, H), lambda i: (0, 0)),
                  pl.BlockSpec((1, 1), lambda i: (0, 0))],
        out_specs=pl.BlockSpec((ROW_BLK, 1), lambda i: (i, 0)),
        out_shape=jax.ShapeDtypeStruct((n, 1), jnp.float32),
    )(h, p0, p1, ws, wn, w2, b2.reshape(1, 1))


NBUF = 2
IB = 16


def _make_seg_sum(steps_per_worker):
    rows_per_tile = AGG_ROWS // 16
    assert steps_per_worker % IB == 0 and IB % NBUF == 0
    nblk = steps_per_worker // IB
    mesh = plsc.VectorSubcoreMesh(core_axis_name="c", subcore_axis_name="s")

    @functools.partial(
        pl.kernel,
        mesh=mesh,
        out_type=jax.ShapeDtypeStruct((2 * AGG_ROWS, H), jnp.float32),
        scratch_types=(
            [pltpu.VMEM((IB, CHUNK), jnp.int32)] * 4 +
            [pltpu.VMEM((CHUNK, H), jnp.float32)] * NBUF +
            [pltpu.VMEM_SHARED((AGG_ROWS, H), jnp.float32)] +
            [pltpu.SemaphoreType.DMA] * (2 + 2 * NBUF)
        ),
    )
    def seg_sum(h_hbm, src_hbm, dst_hbm, zeros_hbm, out_hbm, *rest):
        srcb = rest[0:2]
        dstb = rest[2:4]
        rows = rest[4:4 + NBUF]
        agg_sh = rest[4 + NBUF]
        sem_i = rest[5 + NBUF:7 + NBUF]
        sem_g = rest[7 + NBUF:7 + NBUF + NBUF]
        sem_s = rest[7 + NBUF + NBUF:]
        c = lax.axis_index("c")
        s = lax.axis_index("s")
        wid = c * 16 + s

        pltpu.sync_copy(zeros_hbm, rows[0])
        for z in range(rows_per_tile // ZERO_BLK):
            pltpu.sync_copy(
                rows[0].at[pl.ds(0, ZERO_BLK)],
                agg_sh.at[pl.ds(s * rows_per_tile + z * ZERO_BLK, ZERO_BLK)])

        pltpu.async_copy(src_hbm.at[wid, pl.ds(0, IB)], srcb[0], sem_i[0])
        pltpu.async_copy(dst_hbm.at[wid, pl.ds(0, IB)], dstb[0], sem_i[0])
        plsc.subcore_barrier()

        for blk in range(nblk):
            par = blk % 2
            sv, dv = srcb[par], dstb[par]
            pltpu.make_async_copy(
                src_hbm.at[wid, pl.ds(blk * IB, IB)], sv, sem_i[par]).wait()
            pltpu.make_async_copy(
                dst_hbm.at[wid, pl.ds(blk * IB, IB)], dv, sem_i[par]).wait()
            if blk + 1 < nblk:
                nxt = 1 - par
                pltpu.async_copy(
                    src_hbm.at[wid, pl.ds((blk + 1) * IB, IB)],
                    srcb[nxt], sem_i[nxt])
                pltpu.async_copy(
                    dst_hbm.at[wid, pl.ds((blk + 1) * IB, IB)],
                    dstb[nxt], sem_i[nxt])

            for b in range(NBUF):
                pltpu.async_copy(h_hbm.at[sv.at[b]], rows[b], sem_g[b])

            def body(r, carry, sv=sv, dv=dv):
                base = r * NBUF
                for b in range(NBUF):
                    pltpu.make_async_copy(
                        h_hbm.at[sv.at[base + b]], rows[b], sem_g[b]).wait()
                    pltpu.async_copy(
                        rows[b], agg_sh.at[dv.at[base + b]], sem_s[b],
                        add=True)
                for b in range(NBUF):
                    pltpu.make_async_copy(
                        rows[b], agg_sh.at[dv.at[base + b]], sem_s[b]).wait()

                    @pl.when(r < IB // NBUF - 1)
                    def _():
                        pltpu.async_copy(
                            h_hbm.at[sv.at[base + NBUF + b]], rows[b],
                            sem_g[b])
                return carry

            lax.fori_loop(0, IB // NBUF, body, 0)

        plsc.subcore_barrier()

        for z in range(rows_per_tile // ZERO_BLK):
            off = s * rows_per_tile + z * ZERO_BLK
            pltpu.sync_copy(
                agg_sh.at[pl.ds(off, ZERO_BLK)],
                out_hbm.at[pl.ds(c * AGG_ROWS + off, ZERO_BLK)])

    return seg_sum


def kernel(x, edge_index, W_emb1, b_emb1, Ws1, Wn1, Ws2, Wn2, W_emb2, b_emb2):
    e = edge_index.shape[1]
    pad = (-e) % (NUM_WORKERS * CHUNK * IB)
    src = jnp.concatenate(
        [edge_index[0], jnp.zeros((pad,), jnp.int32)])
    dst = jnp.concatenate(
        [edge_index[1], jnp.full((pad,), AGG_ROWS - 1, jnp.int32)])
    steps = (e + pad) // (NUM_WORKERS * CHUNK)
    src3 = src.reshape(NUM_WORKERS, steps, CHUNK)
    dst3 = dst.reshape(NUM_WORKERS, steps, CHUNK)
    zeros = jnp.zeros((ZERO_BLK, H), jnp.float32)

    seg_sum = _make_seg_sum(steps)

    h0 = _emb1(x, W_emb1, b_emb1)
    parts1 = seg_sum(h0, src3, dst3, zeros)
    h1 = _layer(h0, parts1[:N], parts1[AGG_ROWS:AGG_ROWS + N], Ws1, Wn1)
    parts2 = seg_sum(h1, src3, dst3, zeros)
    return _final(h1, parts2[:N], parts2[AGG_ROWS:AGG_ROWS + N],
                  Ws2, Wn2, W_emb2, b_emb2)


# speedup vs baseline: 3.0887x; 3.0887x over previous
import functools

import jax
import jax.numpy as jnp
from jax import lax
from jax.experimental import pallas as pl
from jax.experimental.pallas import tpu as pltpu
from jax.experimental.pallas import tpu_sc as plsc

N = 10000
H = 128
NUM_WORKERS = 32
CHUNK = 128
ROW_BLK = 1000
AGG_ROWS = 10240
ZERO_BLK = 128


def _gelu(v):
    return 0.5 * v * (1.0 + lax.erf(v * 0.7071067811865475))


def _mm(a, b_t):
    return lax.dot_general(a, b_t, (((1,), (1,)), ((), ())),
                           preferred_element_type=jnp.float32)


def _emb1_body(x_ref, w_ref, b_ref, o_ref):
    o_ref[...] = _gelu(_mm(x_ref[...], w_ref[...]) + b_ref[...])


def _emb1(x, w, b):
    n = x.shape[0]
    grid = (n // ROW_BLK,)
    return pl.pallas_call(
        _emb1_body,
        grid=grid,
        in_specs=[
            pl.BlockSpec((ROW_BLK, H), lambda i: (i, 0)),
            pl.BlockSpec((H, H), lambda i: (0, 0)),
            pl.BlockSpec((1, H), lambda i: (0, 0)),
        ],
        out_specs=pl.BlockSpec((ROW_BLK, H), lambda i: (i, 0)),
        out_shape=jax.ShapeDtypeStruct((n, H), jnp.float32),
    )(x, w, b.reshape(1, H))


def _layer_body(h_ref, p0_ref, p1_ref, ws_ref, wn_ref, o_ref):
    h = h_ref[...]
    agg = p0_ref[...] + p1_ref[...]
    o_ref[...] = _gelu(_mm(h, ws_ref[...]) + _mm(agg, wn_ref[...])) + h


def _layer(h, p0, p1, ws, wn):
    n = h.shape[0]
    grid = (n // ROW_BLK,)
    blk = pl.BlockSpec((ROW_BLK, H), lambda i: (i, 0))
    wblk = pl.BlockSpec((H, H), lambda i: (0, 0))
    return pl.pallas_call(
        _layer_body,
        grid=grid,
        in_specs=[blk, blk, blk, wblk, wblk],
        out_specs=blk,
        out_shape=jax.ShapeDtypeStruct((n, H), jnp.float32),
    )(h, p0, p1, ws, wn)


def _final_body(h_ref, p0_ref, p1_ref, ws_ref, wn_ref, w2_ref, b2_ref, o_ref):
    h = h_ref[...]
    agg = p0_ref[...] + p1_ref[...]
    h2 = _gelu(_mm(h, ws_ref[...]) + _mm(agg, wn_ref[...])) + h
    proj = jnp.sum(h2 * w2_ref[...], axis=1, keepdims=True) + b2_ref[0, 0]
    o_ref[...] = jnp.tanh(proj)


def _final(h, p0, p1, ws, wn, w2, b2):
    n = h.shape[0]
    grid = (n // ROW_BLK,)
    blk = pl.BlockSpec((ROW_BLK, H), lambda i: (i, 0))
    wblk = pl.BlockSpec((H, H), lambda i: (0, 0))
    return pl.pallas_call(
        _final_body,
        grid=grid,
        in_specs=[blk, blk, blk, wblk, wblk,
                  pl.BlockSpec((1, H), lambda i: (0, 0)),
                  pl.BlockSpec((1, 1), lambda i: (0, 0))],
        out_specs=pl.BlockSpec((ROW_BLK, 1), lambda i: (i, 0)),
        out_shape=jax.ShapeDtypeStruct((n, 1), jnp.float32),
    )(h, p0, p1, ws, wn, w2, b2.reshape(1, 1))


NBUF = 2
IB = 16


def _make_seg_sum(steps_per_worker):
    rows_per_tile = AGG_ROWS // 16
    assert steps_per_worker % IB == 0 and IB % NBUF == 0
    nblk = steps_per_worker // IB
    mesh = plsc.VectorSubcoreMesh(core_axis_name="c", subcore_axis_name="s")

    @functools.partial(
        pl.kernel,
        mesh=mesh,
        out_type=jax.ShapeDtypeStruct((2 * AGG_ROWS, H), jnp.float32),
        scratch_types=(
            [pltpu.VMEM((IB, CHUNK), jnp.int32)] * 4 +
            [pltpu.VMEM((CHUNK, H), jnp.float32)] * NBUF +
            [pltpu.VMEM_SHARED((AGG_ROWS, H), jnp.float32)] +
            [pltpu.SemaphoreType.DMA] * (2 + 2 * NBUF)
        ),
    )
    def seg_sum(h_hbm, src_hbm, dst_hbm, zeros_hbm, out_hbm, *rest):
        srcb = rest[0:2]
        dstb = rest[2:4]
        rows = rest[4:4 + NBUF]
        agg_sh = rest[4 + NBUF]
        sem_i = rest[5 + NBUF:7 + NBUF]
        sem_g = rest[7 + NBUF:7 + NBUF + NBUF]
        sem_s = rest[7 + NBUF + NBUF:]
        c = lax.axis_index("c")
        s = lax.axis_index("s")
        wid = c * 16 + s

        pltpu.sync_copy(zeros_hbm, rows[0])
        for z in range(rows_per_tile // ZERO_BLK):
            pltpu.sync_copy(
                rows[0].at[pl.ds(0, ZERO_BLK)],
                agg_sh.at[pl.ds(s * rows_per_tile + z * ZERO_BLK, ZERO_BLK)])

        pltpu.async_copy(src_hbm.at[wid, pl.ds(0, IB)], srcb[0], sem_i[0])
        pltpu.async_copy(dst_hbm.at[wid, pl.ds(0, IB)], dstb[0], sem_i[0])
        plsc.subcore_barrier()

        for blk in range(nblk):
            par = blk % 2
            sv, dv = srcb[par], dstb[par]
            pltpu.make_async_copy(
                src_hbm.at[wid, pl.ds(blk * IB, IB)], sv, sem_i[par]).wait()
            pltpu.make_async_copy(
                dst_hbm.at[wid, pl.ds(blk * IB, IB)], dv, sem_i[par]).wait()
            if blk + 1 < nblk:
                nxt = 1 - par
                pltpu.async_copy(
                    src_hbm.at[wid, pl.ds((blk + 1) * IB, IB)],
                    srcb[nxt], sem_i[nxt])
                pltpu.async_copy(
                    dst_hbm.at[wid, pl.ds((blk + 1) * IB, IB)],
                    dstb[nxt], sem_i[nxt])

            for b in range(NBUF):
                pltpu.async_copy(h_hbm.at[sv.at[b]], rows[b], sem_g[b])

            def body(r, carry, sv=sv, dv=dv):
                base = r * NBUF
                for b in range(NBUF):
                    pltpu.make_async_copy(
                        h_hbm.at[sv.at[base + b]], rows[b], sem_g[b]).wait()
                    pltpu.async_copy(
                        rows[b], agg_sh.at[dv.at[base + b]], sem_s[b],
                        add=True)
                for b in range(NBUF):
                    pltpu.make_async_copy(
                        rows[b], agg_sh.at[dv.at[base + b]], sem_s[b]).wait()

                    @pl.when(r < IB // NBUF - 1)
                    def _():
                        pltpu.async_copy(
                            h_hbm.at[sv.at[base + NBUF + b]], rows[b],
                            sem_g[b])
                return carry

            lax.fori_loop(0, IB // NBUF, body, 0)

        plsc.subcore_barrier()

        for z in range(rows_per_tile // ZERO_BLK):
            off = s * rows_per_tile + z * ZERO_BLK
            pltpu.sync_copy(
                agg_sh.at[pl.ds(off, ZERO_BLK)],
                out_hbm.at[pl.ds(c * AGG_ROWS + off, ZERO_BLK)])

    return seg_sum


def kernel(x, edge_index, W_emb1, b_emb1, Ws1, Wn1, Ws2, Wn2, W_emb2, b_emb2):
    e = edge_index.shape[1]
    pad = (-e) % (NUM_WORKERS * CHUNK * IB)
    pad_ids = jnp.arange(pad, dtype=jnp.int32)
    src = jnp.concatenate([edge_index[0], pad_ids % N])
    dst = jnp.concatenate([edge_index[1], N + pad_ids % (AGG_ROWS - N)])
    steps = (e + pad) // (NUM_WORKERS * CHUNK)
    src3 = src.reshape(NUM_WORKERS, steps, CHUNK)
    dst3 = dst.reshape(NUM_WORKERS, steps, CHUNK)
    zeros = jnp.zeros((ZERO_BLK, H), jnp.float32)

    seg_sum = _make_seg_sum(steps)

    h0 = _emb1(x, W_emb1, b_emb1)
    parts1 = seg_sum(h0, src3, dst3, zeros)
    h1 = _layer(h0, parts1[:N], parts1[AGG_ROWS:AGG_ROWS + N], Ws1, Wn1)
    parts2 = seg_sum(h1, src3, dst3, zeros)
    return _final(h1, parts2[:N], parts2[AGG_ROWS:AGG_ROWS + N],
                  Ws2, Wn2, W_emb2, b_emb2)


# speedup vs baseline: 3.1930x; 1.0338x over previous
import functools

import jax
import jax.numpy as jnp
from jax import lax
from jax.experimental import pallas as pl
from jax.experimental.pallas import tpu as pltpu
from jax.experimental.pallas import tpu_sc as plsc

N = 10000
H = 128
NUM_WORKERS = 32
CHUNK = 128
ROW_BLK = 1000
AGG_ROWS = 10240
ZERO_BLK = 128


def _gelu(v):
    return 0.5 * v * (1.0 + lax.erf(v * 0.7071067811865475))


def _mm(a, b_t):
    return lax.dot_general(a, b_t, (((1,), (1,)), ((), ())),
                           preferred_element_type=jnp.float32)


def _emb1_body(x_ref, w_ref, b_ref, o_ref):
    o_ref[...] = _gelu(_mm(x_ref[...], w_ref[...]) + b_ref[...])


def _emb1(x, w, b):
    n = x.shape[0]
    grid = (n // ROW_BLK,)
    return pl.pallas_call(
        _emb1_body,
        grid=grid,
        in_specs=[
            pl.BlockSpec((ROW_BLK, H), lambda i: (i, 0)),
            pl.BlockSpec((H, H), lambda i: (0, 0)),
            pl.BlockSpec((1, H), lambda i: (0, 0)),
        ],
        out_specs=pl.BlockSpec((ROW_BLK, H), lambda i: (i, 0)),
        out_shape=jax.ShapeDtypeStruct((n, H), jnp.float32),
    )(x, w, b.reshape(1, H))


def _mmT_body(h_ref, w_ref, o_ref):
    o_ref[...] = _mm(h_ref[...], w_ref[...])


def _mmT(h, w):
    n = h.shape[0]
    return pl.pallas_call(
        _mmT_body,
        grid=(n // ROW_BLK,),
        in_specs=[pl.BlockSpec((ROW_BLK, H), lambda i: (i, 0)),
                  pl.BlockSpec((H, H), lambda i: (0, 0))],
        out_specs=pl.BlockSpec((ROW_BLK, H), lambda i: (i, 0)),
        out_shape=jax.ShapeDtypeStruct((n, H), jnp.float32),
    )(h, w)


def _pspec():
    return [pl.BlockSpec((1, ROW_BLK, H), lambda i: (0, i, 0)),
            pl.BlockSpec((1, ROW_BLK, H), lambda i: (1, i, 0))]


def _layer_body(h_ref, hs_ref, p0_ref, p1_ref, wn_ref, o_ref):
    agg = p0_ref[0] + p1_ref[0]
    o_ref[...] = _gelu(hs_ref[...] + _mm(agg, wn_ref[...])) + h_ref[...]


def _layer(h, hs, parts, wn):
    n = h.shape[0]
    grid = (n // ROW_BLK,)
    blk = pl.BlockSpec((ROW_BLK, H), lambda i: (i, 0))
    return pl.pallas_call(
        _layer_body,
        grid=grid,
        in_specs=[blk, blk] + _pspec() + [pl.BlockSpec((H, H), lambda i: (0, 0))],
        out_specs=blk,
        out_shape=jax.ShapeDtypeStruct((n, H), jnp.float32),
    )(h, hs, parts, parts, wn)


def _final_body(h_ref, hs_ref, p0_ref, p1_ref, wn_ref, w2_ref, b2_ref, o_ref):
    agg = p0_ref[0] + p1_ref[0]
    h2 = _gelu(hs_ref[...] + _mm(agg, wn_ref[...])) + h_ref[...]
    proj = jnp.sum(h2 * w2_ref[...], axis=1, keepdims=True) + b2_ref[0, 0]
    o_ref[...] = jnp.tanh(proj)


def _final(h, hs, parts, wn, w2, b2):
    n = h.shape[0]
    grid = (n // ROW_BLK,)
    blk = pl.BlockSpec((ROW_BLK, H), lambda i: (i, 0))
    return pl.pallas_call(
        _final_body,
        grid=grid,
        in_specs=[blk, blk] + _pspec() + [
            pl.BlockSpec((H, H), lambda i: (0, 0)),
            pl.BlockSpec((1, H), lambda i: (0, 0)),
            pl.BlockSpec((1, 1), lambda i: (0, 0))],
        out_specs=pl.BlockSpec((ROW_BLK, 1), lambda i: (i, 0)),
        out_shape=jax.ShapeDtypeStruct((n, 1), jnp.float32),
    )(h, hs, parts, parts, wn, w2, b2.reshape(1, 1))


NBUF = 2
IB = 16


def _make_seg_sum(steps_per_worker):
    rows_per_tile = AGG_ROWS // 16
    assert steps_per_worker % IB == 0 and IB % NBUF == 0
    nblk = steps_per_worker // IB
    mesh = plsc.VectorSubcoreMesh(core_axis_name="c", subcore_axis_name="s")

    @functools.partial(
        pl.kernel,
        mesh=mesh,
        out_type=jax.ShapeDtypeStruct((2 * AGG_ROWS, H), jnp.float32),
        scratch_types=(
            [pltpu.VMEM((IB, CHUNK), jnp.int32)] * 4 +
            [pltpu.VMEM((CHUNK, H), jnp.float32)] * NBUF +
            [pltpu.VMEM_SHARED((AGG_ROWS, H), jnp.float32)] +
            [pltpu.SemaphoreType.DMA] * (2 + 2 * NBUF)
        ),
    )
    def seg_sum(h_hbm, src_hbm, dst_hbm, zeros_hbm, out_hbm, *rest):
        srcb = rest[0:2]
        dstb = rest[2:4]
        rows = rest[4:4 + NBUF]
        agg_sh = rest[4 + NBUF]
        sem_i = rest[5 + NBUF:7 + NBUF]
        sem_g = rest[7 + NBUF:7 + NBUF + NBUF]
        sem_s = rest[7 + NBUF + NBUF:]
        c = lax.axis_index("c")
        s = lax.axis_index("s")
        wid = c * 16 + s

        pltpu.sync_copy(zeros_hbm, rows[0])
        for z in range(rows_per_tile // ZERO_BLK):
            pltpu.sync_copy(
                rows[0].at[pl.ds(0, ZERO_BLK)],
                agg_sh.at[pl.ds(s * rows_per_tile + z * ZERO_BLK, ZERO_BLK)])

        pltpu.async_copy(src_hbm.at[wid, pl.ds(0, IB)], srcb[0], sem_i[0])
        pltpu.async_copy(dst_hbm.at[wid, pl.ds(0, IB)], dstb[0], sem_i[0])
        plsc.subcore_barrier()

        for blk in range(nblk):
            par = blk % 2
            sv, dv = srcb[par], dstb[par]
            pltpu.make_async_copy(
                src_hbm.at[wid, pl.ds(blk * IB, IB)], sv, sem_i[par]).wait()
            pltpu.make_async_copy(
                dst_hbm.at[wid, pl.ds(blk * IB, IB)], dv, sem_i[par]).wait()
            if blk + 1 < nblk:
                nxt = 1 - par
                pltpu.async_copy(
                    src_hbm.at[wid, pl.ds((blk + 1) * IB, IB)],
                    srcb[nxt], sem_i[nxt])
                pltpu.async_copy(
                    dst_hbm.at[wid, pl.ds((blk + 1) * IB, IB)],
                    dstb[nxt], sem_i[nxt])

            for b in range(NBUF):
                pltpu.async_copy(h_hbm.at[sv.at[b]], rows[b], sem_g[b])

            def body(r, carry, sv=sv, dv=dv):
                base = r * NBUF
                for b in range(NBUF):
                    pltpu.make_async_copy(
                        h_hbm.at[sv.at[base + b]], rows[b], sem_g[b]).wait()
                    pltpu.async_copy(
                        rows[b], agg_sh.at[dv.at[base + b]], sem_s[b],
                        add=True)
                for b in range(NBUF):
                    pltpu.make_async_copy(
                        rows[b], agg_sh.at[dv.at[base + b]], sem_s[b]).wait()

                    @pl.when(r < IB // NBUF - 1)
                    def _():
                        pltpu.async_copy(
                            h_hbm.at[sv.at[base + NBUF + b]], rows[b],
                            sem_g[b])
                return carry

            lax.fori_loop(0, IB // NBUF, body, 0)

        plsc.subcore_barrier()

        for z in range(rows_per_tile // ZERO_BLK):
            off = s * rows_per_tile + z * ZERO_BLK
            pltpu.sync_copy(
                agg_sh.at[pl.ds(off, ZERO_BLK)],
                out_hbm.at[pl.ds(c * AGG_ROWS + off, ZERO_BLK)])

    return seg_sum


def kernel(x, edge_index, W_emb1, b_emb1, Ws1, Wn1, Ws2, Wn2, W_emb2, b_emb2):
    e = edge_index.shape[1]
    pad = (-e) % (NUM_WORKERS * CHUNK * IB)
    pad_ids = jnp.arange(pad, dtype=jnp.int32)
    src = jnp.concatenate([edge_index[0], pad_ids % N])
    dst = jnp.concatenate([edge_index[1], N + pad_ids % (AGG_ROWS - N)])
    steps = (e + pad) // (NUM_WORKERS * CHUNK)
    src3 = src.reshape(NUM_WORKERS, steps, CHUNK)
    dst3 = dst.reshape(NUM_WORKERS, steps, CHUNK)
    zeros = jnp.zeros((ZERO_BLK, H), jnp.float32)

    seg_sum = _make_seg_sum(steps)

    h0 = _emb1(x, W_emb1, b_emb1)
    parts1 = seg_sum(h0, src3, dst3, zeros).reshape(2, AGG_ROWS, H)
    hs1 = _mmT(h0, Ws1)
    h1 = _layer(h0, hs1, parts1, Wn1)
    parts2 = seg_sum(h1, src3, dst3, zeros).reshape(2, AGG_ROWS, H)
    hs2 = _mmT(h1, Ws2)
    return _final(h1, hs2, parts2, Wn2, W_emb2, b_emb2)
